# baseline (speedup 1.0000x reference)
import jax
import jax.numpy as jnp
import numpy as np
from jax.experimental import pallas as pl

B = 1024
L = 20
C = 23
HID = 128
NH = 4
DH = HID // NH
VOCAB = 23
POS = 200
ITER = 1000
LMAX = 10000.0
LMIN = 0.001

BB = 8
LP = 24
LC = L * C

_INTERPRET = False


def _div_term_np():
    base = LMAX / (2.0 * np.pi)
    scale = LMIN / LMAX
    return (base * scale ** (np.arange(0, DH, 2, dtype=np.float32) / DH)).astype(np.float32)


def _inv_dt4_np():
    dt = _div_term_np()
    inv = (np.float32(1.0) / dt).astype(np.float32)
    h = (inv / 2.0).astype(np.float32)
    full = np.stack([np.concatenate([inv, inv]), np.concatenate([inv, inv]),
                     np.concatenate([h, h]), np.concatenate([h, h])])
    return full.reshape(NH, DH)


def _onehot(idx, n):
    iota = jax.lax.broadcasted_iota(jnp.int32, (1, n), 1)
    return (idx == iota).astype(jnp.float32)


def _body(sqF_r, spF_r, siF_r, a2_r, caF_r, arep_r, cbt_r, st_r, pt_r, it_r, inv4_r,
          seq_e_r, seq_me_r, cand_e_r, cand_me_r):
    st = st_r[...]
    inv4 = inv4_r[...]
    lane = jax.lax.broadcasted_iota(jnp.int32, (1, 1, 1, DH), 3)
    sin_mask = lane < (DH // 2)

    sp = jnp.minimum(spF_r[...], POS - 1)
    si = jnp.minimum(siF_r[...], ITER - 1)
    pi_flat = (jnp.dot(_onehot(sp, POS), pt_r[...], preferred_element_type=jnp.float32)
               + jnp.dot(_onehot(si, ITER), it_r[...], preferred_element_type=jnp.float32))
    es_flat = jnp.dot(_onehot(sqF_r[...], VOCAB), st, preferred_element_type=jnp.float32)
    sc_flat = jnp.dot(_onehot(caF_r[...], VOCAB), st, preferred_element_type=jnp.float32)

    j_iota = jax.lax.broadcasted_iota(jnp.int32, (LC, LP), 0)
    k_iota = jax.lax.broadcasted_iota(jnp.int32, (LC, LP), 1)
    Rl = ((j_iota // C) == k_iota).astype(jnp.float32)
    Rc = ((j_iota % C) == k_iota).astype(jnp.float32)

    for b in range(BB):
        pb = pi_flat[b * LP:(b + 1) * LP]
        eb = es_flat[b * LP:(b + 1) * LP]
        scb = sc_flat[b * LP:(b + 1) * LP]
        seq_e_r[b] = (eb + pb)[0:L]
        cand_e_r[b] = (jnp.dot(Rl, pb, preferred_element_type=jnp.float32)
                       + jnp.dot(Rc, scb, preferred_element_type=jnp.float32))

    inv4b = inv4[None, None]
    xs = a2_r[...][:, :, None, None] * inv4b
    seq_me_r[...] = jnp.where(sin_mask, jnp.sin(xs), jnp.cos(xs))

    cm = arep_r[...] + cbt_r[...]
    CH = LC // 5
    for s in range(0, LC, CH):
        xc = cm[:, s:s + CH, None, None] * inv4b
        cand_me_r[:, s:s + CH] = jnp.where(sin_mask, jnp.sin(xc), jnp.cos(xc))


@jax.jit
def _run(sqF, spF, siF, a2, caF, arep, cbt, seq_table, pos_table, iter_table, inv4):
    grid = (B // BB,)
    out_shapes = [
        jax.ShapeDtypeStruct((B, L, HID), jnp.float32),
        jax.ShapeDtypeStruct((B, L, NH, DH), jnp.float32),
        jax.ShapeDtypeStruct((B, LC, HID), jnp.float32),
        jax.ShapeDtypeStruct((B, LC, NH, DH), jnp.float32),
    ]
    flat = pl.BlockSpec((BB * LP, 1), lambda i: (i, 0))
    in_specs = [
        flat, flat, flat,
        pl.BlockSpec((BB, L), lambda i: (i, 0)),
        flat,
        pl.BlockSpec((BB, LC), lambda i: (i, 0)),
        pl.BlockSpec((BB, LC), lambda i: (i, 0)),
        pl.BlockSpec((VOCAB, HID), lambda i: (0, 0)),
        pl.BlockSpec((POS, HID), lambda i: (0, 0)),
        pl.BlockSpec((ITER, HID), lambda i: (0, 0)),
        pl.BlockSpec((NH, DH), lambda i: (0, 0)),
    ]
    out_specs = [
        pl.BlockSpec((BB, L, HID), lambda i: (i, 0, 0)),
        pl.BlockSpec((BB, L, NH, DH), lambda i: (i, 0, 0, 0)),
        pl.BlockSpec((BB, LC, HID), lambda i: (i, 0, 0)),
        pl.BlockSpec((BB, LC, NH, DH), lambda i: (i, 0, 0, 0)),
    ]
    return pl.pallas_call(
        _body,
        grid=grid,
        in_specs=in_specs,
        out_specs=out_specs,
        out_shape=out_shapes,
        interpret=_INTERPRET,
    )(sqF, spF, siF, a2, caF, arep, cbt, seq_table, pos_table, iter_table, inv4)


def _pad_flat(x2d, dtype):
    xp = jnp.pad(x2d.astype(dtype), ((0, 0), (0, LP - x2d.shape[1])))
    return xp.reshape(B * LP, 1)


def kernel(seq, seq_pos, seq_iter, seq_mass_forward, candidate_aa, candidate_aa_mass,
           seq_table, pos_table, iter_table):
    inv4 = jnp.asarray(_inv_dt4_np())
    sqF = _pad_flat(seq, jnp.int32)
    spF = _pad_flat(seq_pos, jnp.int32)
    siF = _pad_flat(seq_iter, jnp.int32)
    caF = _pad_flat(candidate_aa, jnp.int32)
    arep = jnp.repeat(seq_mass_forward, C, axis=1)
    cbt = jnp.tile(candidate_aa_mass, (1, L))
    return tuple(_run(sqF, spF, siF, seq_mass_forward.astype(jnp.float32), caF, arep, cbt,
                      seq_table, pos_table, iter_table, inv4))


# speedup vs baseline: 3.1917x; 3.1917x over previous
import jax
import jax.numpy as jnp
import numpy as np
from jax.experimental import pallas as pl

B = 1024
L = 20
C = 23
HID = 128
NH = 4
DH = HID // NH
VOCAB = 23
POS = 200
ITER = 1000
LMAX = 10000.0
LMIN = 0.001

BB = 8
LP = 24
LC = L * C

_INTERPRET = False


def _div_term_np():
    base = LMAX / (2.0 * np.pi)
    scale = LMIN / LMAX
    return (base * scale ** (np.arange(0, DH, 2, dtype=np.float32) / DH)).astype(np.float32)


def _inv_dt4_np():
    dt = _div_term_np()
    inv = (np.float32(1.0) / dt).astype(np.float32)
    h = (inv / 2.0).astype(np.float32)
    return np.concatenate([inv, inv, inv, inv, h, h, h, h]).reshape(1, HID)


def _onehot(idx, n):
    iota = jax.lax.broadcasted_iota(jnp.int32, (1, n), 1)
    return (idx == iota).astype(jnp.float32)


def _body(sqF_r, spF_r, siF_r, a2_r, caF_r, arep_r, cbt_r, st_r, pt_r, it_r, inv4_r,
          seq_e_r, seq_me_r, cand_e_r, cand_me_r):
    st = st_r[...]
    inv128 = inv4_r[...][None]
    lane = jax.lax.broadcasted_iota(jnp.int32, (1, 1, HID), 2)
    sin_mask = (lane % DH) < (DH // 2)

    sp = jnp.minimum(spF_r[...], POS - 1)
    si = jnp.minimum(siF_r[...], ITER - 1)
    pi_flat = (jnp.dot(_onehot(sp, POS), pt_r[...], preferred_element_type=jnp.float32)
               + jnp.dot(_onehot(si, ITER), it_r[...], preferred_element_type=jnp.float32))
    es_flat = jnp.dot(_onehot(sqF_r[...], VOCAB), st, preferred_element_type=jnp.float32)
    sc_flat = jnp.dot(_onehot(caF_r[...], VOCAB), st, preferred_element_type=jnp.float32)

    j_iota = jax.lax.broadcasted_iota(jnp.int32, (LC, LP), 0)
    k_iota = jax.lax.broadcasted_iota(jnp.int32, (LC, LP), 1)
    Rl = ((j_iota // C) == k_iota).astype(jnp.float32)
    Rc = ((j_iota % C) == k_iota).astype(jnp.float32)

    for b in range(BB):
        pb = pi_flat[b * LP:(b + 1) * LP]
        eb = es_flat[b * LP:(b + 1) * LP]
        scb = sc_flat[b * LP:(b + 1) * LP]
        seq_e_r[b] = (eb + pb)[0:L]
        cand_e_r[b] = (jnp.dot(Rl, pb, preferred_element_type=jnp.float32)
                       + jnp.dot(Rc, scb, preferred_element_type=jnp.float32))

    def heads4(out):
        r, k = out.shape[0], out.shape[1]
        return jnp.concatenate(
            [jax.lax.broadcast_in_dim(out[:, :, h * DH:(h + 1) * DH],
                                      (r, k, 1, DH), (0, 1, 3)) for h in range(NH)],
            axis=2)

    xs = a2_r[...][:, :, None] * inv128
    outs = jnp.where(sin_mask, jnp.sin(xs), jnp.cos(xs))
    seq_me_r[...] = heads4(outs)

    cm3 = (arep_r[...] + cbt_r[...])[:, :, None]
    CH = 96
    for s in range(0, LC, CH):
        e = min(s + CH, LC)
        xc = cm3[:, s:e] * inv128
        outc = jnp.where(sin_mask, jnp.sin(xc), jnp.cos(xc))
        cand_me_r[:, s:e] = heads4(outc)


@jax.jit
def _run(sqF, spF, siF, a2, caF, arep, cbt, seq_table, pos_table, iter_table, inv4):
    grid = (B // BB,)
    out_shapes = [
        jax.ShapeDtypeStruct((B, L, HID), jnp.float32),
        jax.ShapeDtypeStruct((B, L, NH, DH), jnp.float32),
        jax.ShapeDtypeStruct((B, LC, HID), jnp.float32),
        jax.ShapeDtypeStruct((B, LC, NH, DH), jnp.float32),
    ]
    flat = pl.BlockSpec((BB * LP, 1), lambda i: (i, 0))
    in_specs = [
        flat, flat, flat,
        pl.BlockSpec((BB, L), lambda i: (i, 0)),
        flat,
        pl.BlockSpec((BB, LC), lambda i: (i, 0)),
        pl.BlockSpec((BB, LC), lambda i: (i, 0)),
        pl.BlockSpec((VOCAB, HID), lambda i: (0, 0)),
        pl.BlockSpec((POS, HID), lambda i: (0, 0)),
        pl.BlockSpec((ITER, HID), lambda i: (0, 0)),
        pl.BlockSpec((1, HID), lambda i: (0, 0)),
    ]
    out_specs = [
        pl.BlockSpec((BB, L, HID), lambda i: (i, 0, 0)),
        pl.BlockSpec((BB, L, NH, DH), lambda i: (i, 0, 0, 0)),
        pl.BlockSpec((BB, LC, HID), lambda i: (i, 0, 0)),
        pl.BlockSpec((BB, LC, NH, DH), lambda i: (i, 0, 0, 0)),
    ]
    return pl.pallas_call(
        _body,
        grid=grid,
        in_specs=in_specs,
        out_specs=out_specs,
        out_shape=out_shapes,
        interpret=_INTERPRET,
    )(sqF, spF, siF, a2, caF, arep, cbt, seq_table, pos_table, iter_table, inv4)


def _pad_flat(x2d, dtype):
    xp = jnp.pad(x2d.astype(dtype), ((0, 0), (0, LP - x2d.shape[1])))
    return xp.reshape(B * LP, 1)


def kernel(seq, seq_pos, seq_iter, seq_mass_forward, candidate_aa, candidate_aa_mass,
           seq_table, pos_table, iter_table):
    inv4 = jnp.asarray(_inv_dt4_np())
    sqF = _pad_flat(seq, jnp.int32)
    spF = _pad_flat(seq_pos, jnp.int32)
    siF = _pad_flat(seq_iter, jnp.int32)
    caF = _pad_flat(candidate_aa, jnp.int32)
    arep = jnp.repeat(seq_mass_forward, C, axis=1)
    cbt = jnp.tile(candidate_aa_mass, (1, L))
    return tuple(_run(sqF, spF, siF, seq_mass_forward.astype(jnp.float32), caF, arep, cbt,
                      seq_table, pos_table, iter_table, inv4))


# speedup vs baseline: 4.4428x; 1.3920x over previous
import jax
import jax.numpy as jnp
import numpy as np
from jax.experimental import pallas as pl

B = 1024
L = 20
C = 23
HID = 128
NH = 4
DH = HID // NH
VOCAB = 23
POS = 200
ITER = 1000
LMAX = 10000.0
LMIN = 0.001

BB = 8
LP = 24
LC = L * C

_INTERPRET = False


def _div_term_np():
    base = LMAX / (2.0 * np.pi)
    scale = LMIN / LMAX
    return (base * scale ** (np.arange(0, DH, 2, dtype=np.float32) / DH)).astype(np.float32)


def _inv_dt4_np():
    dt = _div_term_np()
    inv = (np.float32(1.0) / dt).astype(np.float32)
    h = (inv / 2.0).astype(np.float32)
    return np.concatenate([inv, inv, inv, inv, h, h, h, h]).reshape(1, HID)


def _onehot(idx, n):
    iota = jax.lax.broadcasted_iota(jnp.int32, (1, n), 1)
    return (idx == iota).astype(jnp.float32)


def _body(sqF_r, spF_r, siF_r, a2_r, caF_r, arep_r, cbt_r, st_r, pt_r, it_r, inv4_r,
          seq_e_r, seq_me_r, cand_e_r, cand_me_r):
    st = st_r[...]
    inv128 = inv4_r[...][None]
    lane = jax.lax.broadcasted_iota(jnp.int32, (1, 1, HID), 2)
    sin_mask = (lane % DH) < (DH // 2)

    sp = jnp.minimum(spF_r[...], POS - 1)
    si = jnp.minimum(siF_r[...], ITER - 1)
    pi_flat = (jnp.dot(_onehot(sp, POS), pt_r[...], preferred_element_type=jnp.float32)
               + jnp.dot(_onehot(si, ITER), it_r[...], preferred_element_type=jnp.float32))
    es_flat = jnp.dot(_onehot(sqF_r[...], VOCAB), st, preferred_element_type=jnp.float32)
    sc_flat = jnp.dot(_onehot(caF_r[...], VOCAB), st, preferred_element_type=jnp.float32)

    j_iota = jax.lax.broadcasted_iota(jnp.int32, (LC, LP), 0)
    k_iota = jax.lax.broadcasted_iota(jnp.int32, (LC, LP), 1)
    Rl = ((j_iota // C) == k_iota).astype(jnp.float32)
    Rc = ((j_iota % C) == k_iota).astype(jnp.float32)

    for b in range(BB):
        pb = pi_flat[b * LP:(b + 1) * LP]
        eb = es_flat[b * LP:(b + 1) * LP]
        scb = sc_flat[b * LP:(b + 1) * LP]
        seq_e_r[b] = (eb + pb)[0:L]
        cand_e_r[b] = (jnp.dot(Rl, pb, preferred_element_type=jnp.float32)
                       + jnp.dot(Rc, scb, preferred_element_type=jnp.float32))

    def heads4(out):
        r, k = out.shape[0], out.shape[1]
        return jnp.concatenate(
            [jax.lax.broadcast_in_dim(out[:, :, h * DH:(h + 1) * DH],
                                      (r, k, 2, DH), (0, 1, 3)) for h in (0, 2)],
            axis=2)

    xs = a2_r[...][:, :, None] * inv128
    outs = jnp.where(sin_mask, jnp.sin(xs), jnp.cos(xs))
    seq_me_r[...] = heads4(outs)

    cm3 = (arep_r[...] + cbt_r[...])[:, :, None]
    CH = 96
    for s in range(0, LC, CH):
        e = min(s + CH, LC)
        xc = cm3[:, s:e] * inv128
        outc = jnp.where(sin_mask, jnp.sin(xc), jnp.cos(xc))
        cand_me_r[:, s:e] = heads4(outc)


@jax.jit
def _run(sqF, spF, siF, a2, caF, arep, cbt, seq_table, pos_table, iter_table, inv4):
    grid = (B // BB,)
    out_shapes = [
        jax.ShapeDtypeStruct((B, L, HID), jnp.float32),
        jax.ShapeDtypeStruct((B, L, NH, DH), jnp.float32),
        jax.ShapeDtypeStruct((B, LC, HID), jnp.float32),
        jax.ShapeDtypeStruct((B, LC, NH, DH), jnp.float32),
    ]
    flat = pl.BlockSpec((BB * LP, 1), lambda i: (i, 0))
    in_specs = [
        flat, flat, flat,
        pl.BlockSpec((BB, L), lambda i: (i, 0)),
        flat,
        pl.BlockSpec((BB, LC), lambda i: (i, 0)),
        pl.BlockSpec((BB, LC), lambda i: (i, 0)),
        pl.BlockSpec((VOCAB, HID), lambda i: (0, 0)),
        pl.BlockSpec((POS, HID), lambda i: (0, 0)),
        pl.BlockSpec((ITER, HID), lambda i: (0, 0)),
        pl.BlockSpec((1, HID), lambda i: (0, 0)),
    ]
    out_specs = [
        pl.BlockSpec((BB, L, HID), lambda i: (i, 0, 0)),
        pl.BlockSpec((BB, L, NH, DH), lambda i: (i, 0, 0, 0)),
        pl.BlockSpec((BB, LC, HID), lambda i: (i, 0, 0)),
        pl.BlockSpec((BB, LC, NH, DH), lambda i: (i, 0, 0, 0)),
    ]
    return pl.pallas_call(
        _body,
        grid=grid,
        in_specs=in_specs,
        out_specs=out_specs,
        out_shape=out_shapes,
        interpret=_INTERPRET,
    )(sqF, spF, siF, a2, caF, arep, cbt, seq_table, pos_table, iter_table, inv4)


def _pad_flat(x2d, dtype):
    xp = jnp.pad(x2d.astype(dtype), ((0, 0), (0, LP - x2d.shape[1])))
    return xp.reshape(B * LP, 1)


def kernel(seq, seq_pos, seq_iter, seq_mass_forward, candidate_aa, candidate_aa_mass,
           seq_table, pos_table, iter_table):
    inv4 = jnp.asarray(_inv_dt4_np())
    sqF = _pad_flat(seq, jnp.int32)
    spF = _pad_flat(seq_pos, jnp.int32)
    siF = _pad_flat(seq_iter, jnp.int32)
    caF = _pad_flat(candidate_aa, jnp.int32)
    arep = jnp.repeat(seq_mass_forward, C, axis=1)
    cbt = jnp.tile(candidate_aa_mass, (1, L))
    return tuple(_run(sqF, spF, siF, seq_mass_forward.astype(jnp.float32), caF, arep, cbt,
                      seq_table, pos_table, iter_table, inv4))


# speedup vs baseline: 4.6356x; 1.0434x over previous
import functools

import jax
import jax.numpy as jnp
import numpy as np
from jax import lax
from jax.experimental import pallas as pl
from jax.experimental.pallas import tpu as pltpu, tpu_sc as plsc

B = 1024
L = 20
C = 23
HID = 128
NH = 4
DH = HID // NH
VOCAB = 23
POS = 200
ITER = 1000
LMAX = 10000.0
LMIN = 0.001

BB = 8
LP = 24
LC = L * C
NW = 32
BPW = B // NW

_INTERPRET = False


def _div_term_np():
    base = LMAX / (2.0 * np.pi)
    scale = LMIN / LMAX
    return (base * scale ** (np.arange(0, DH, 2, dtype=np.float32) / DH)).astype(np.float32)


def _inv_dt128_np():
    dt = _div_term_np()
    inv = (np.float32(1.0) / dt).astype(np.float32)
    h = (inv / 2.0).astype(np.float32)
    return np.concatenate([inv, inv, inv, inv, h, h, h, h]).reshape(1, HID)


def _sc_embed(sqP, spP, siP, caP, seq_table, pos_table, iter_table):
    mesh = plsc.VectorSubcoreMesh(core_axis_name="c", subcore_axis_name="s")

    @functools.partial(
        pl.kernel, mesh=mesh,
        out_type=[jax.ShapeDtypeStruct((B, L, HID), jnp.float32),
                  jax.ShapeDtypeStruct((B, LC, HID), jnp.float32)],
        scratch_types=[
            pltpu.VMEM((LP,), jnp.int32),
            pltpu.VMEM((LP,), jnp.int32),
            pltpu.VMEM((LP,), jnp.int32),
            pltpu.VMEM((LP,), jnp.int32),
            pltpu.VMEM((LP, HID), jnp.float32),
            pltpu.VMEM((LP, HID), jnp.float32),
            pltpu.VMEM((LP, HID), jnp.float32),
            pltpu.VMEM((LP, HID), jnp.float32),
            pltpu.VMEM((L, HID), jnp.float32),
            pltpu.VMEM((LC, HID), jnp.float32),
            pltpu.SemaphoreType.DMA,
        ],
    )
    def k(sq_h, sp_h, si_h, ca_h, st_h, pt_h, it_h, seqe_h, cande_h,
          sqv, spv, siv, cav, posr, iterr, seqr, candr, pib, candbuf, sem):
        wid = lax.axis_index("s") * 2 + lax.axis_index("c")

        def per_batch(b, carry):
            row = wid * BPW + b
            pltpu.sync_copy(sq_h.at[row], sqv)
            pltpu.sync_copy(sp_h.at[row], spv)
            pltpu.sync_copy(si_h.at[row], siv)
            pltpu.sync_copy(ca_h.at[row], cav)
            pltpu.async_copy(pt_h.at[spv], posr, sem).wait()
            pltpu.async_copy(it_h.at[siv], iterr, sem).wait()
            pltpu.async_copy(st_h.at[sqv], seqr, sem).wait()
            pltpu.async_copy(st_h.at[cav], candr, sem).wait()
            for l in range(L):
                for j in range(8):
                    sl = pl.ds(j * 16, 16)
                    pib[l, sl] = posr[l, sl] + iterr[l, sl] + seqr[l, sl]

            def per_l(l, acc):
                def per_c(c, acc2):
                    r = l * C + c
                    for j in range(8):
                        sl = pl.ds(j * 16, 16)
                        candbuf[r, sl] = posr[l, sl] + iterr[l, sl] + candr[c, sl]
                    return acc2
                return lax.fori_loop(0, C, per_c, acc)

            lax.fori_loop(0, L, per_l, 0)
            pltpu.sync_copy(pib, seqe_h.at[row])
            pltpu.sync_copy(candbuf, cande_h.at[row])
            return carry

        lax.fori_loop(0, BPW, per_batch, 0)

    return k(sqP, spP, siP, caP, seq_table, pos_table, iter_table)


def _tc_body(a2_r, arep_r, cbt_r, inv_r, seq_me_r, cand_me_r):
    inv128 = inv_r[...][None]
    lane = jax.lax.broadcasted_iota(jnp.int32, (1, 1, HID), 2)
    sin_mask = (lane % DH) < (DH // 2)

    def heads4(out):
        r, k = out.shape[0], out.shape[1]
        return jnp.concatenate(
            [jax.lax.broadcast_in_dim(out[:, :, h * DH:(h + 1) * DH],
                                      (r, k, 2, DH), (0, 1, 3)) for h in (0, 2)],
            axis=2)

    xs = a2_r[...][:, :, None] * inv128
    outs = jnp.where(sin_mask, jnp.sin(xs), jnp.cos(xs))
    seq_me_r[...] = heads4(outs)

    cm3 = (arep_r[...] + cbt_r[...])[:, :, None]
    CH = 96
    for s in range(0, LC, CH):
        e = min(s + CH, LC)
        xc = cm3[:, s:e] * inv128
        outc = jnp.where(sin_mask, jnp.sin(xc), jnp.cos(xc))
        cand_me_r[:, s:e] = heads4(outc)


@jax.jit
def _run(sqP, spP, siP, caP, a2, arep, cbt, seq_table, pos_table, iter_table, inv):
    seq_e, cand_e = _sc_embed(sqP, spP, siP, caP, seq_table, pos_table, iter_table)
    seq_me, cand_me = pl.pallas_call(
        _tc_body,
        grid=(B // BB,),
        in_specs=[
            pl.BlockSpec((BB, L), lambda i: (i, 0)),
            pl.BlockSpec((BB, LC), lambda i: (i, 0)),
            pl.BlockSpec((BB, LC), lambda i: (i, 0)),
            pl.BlockSpec((1, HID), lambda i: (0, 0)),
        ],
        out_specs=[
            pl.BlockSpec((BB, L, NH, DH), lambda i: (i, 0, 0, 0)),
            pl.BlockSpec((BB, LC, NH, DH), lambda i: (i, 0, 0, 0)),
        ],
        out_shape=[
            jax.ShapeDtypeStruct((B, L, NH, DH), jnp.float32),
            jax.ShapeDtypeStruct((B, LC, NH, DH), jnp.float32),
        ],
        interpret=_INTERPRET,
    )(a2, arep, cbt, inv)
    return seq_e, seq_me, cand_e, cand_me


def _pad24(x2d):
    return jnp.pad(x2d.astype(jnp.int32), ((0, 0), (0, LP - x2d.shape[1])))


def kernel(seq, seq_pos, seq_iter, seq_mass_forward, candidate_aa, candidate_aa_mass,
           seq_table, pos_table, iter_table):
    inv = jnp.asarray(_inv_dt128_np())
    sqP = _pad24(seq)
    spP = _pad24(jnp.minimum(seq_pos, POS - 1))
    siP = _pad24(jnp.minimum(seq_iter, ITER - 1))
    caP = _pad24(candidate_aa)
    arep = jnp.repeat(seq_mass_forward, C, axis=1)
    cbt = jnp.tile(candidate_aa_mass, (1, L))
    return tuple(_run(sqP, spP, siP, caP, seq_mass_forward.astype(jnp.float32),
                      arep, cbt, seq_table, pos_table, iter_table, inv))
